# Initial kernel scaffold; baseline (speedup 1.0000x reference)
import jax
import jax.numpy as jnp
from jax.experimental import pallas as pl


def kernel(context_tokens, time_tokens, act_tokens, time_table, act_table, age_table, gender_table, W_time, b_time, W_act, b_act):
    raise NotImplementedError("write your pallas kernel here")


# speedup vs baseline: 3.5452x; 3.5452x over previous
import functools

import jax
import jax.numpy as jnp
from jax import lax
from jax.experimental import pallas as pl
from jax.experimental.pallas import tpu as pltpu
from jax.experimental.pallas import tpu_sc as plsc

_GATHER_W = 128
_ROW_BLOCK = 4000


def _project_body(tt_ref, at_ref, wt_ref, bt_ref, wa_ref, ba_ref,
                  pt_ref, pa_ref):
    pt_ref[...] = jnp.dot(tt_ref[...], wt_ref[...],
                          preferred_element_type=jnp.float32,
                          precision=lax.Precision.HIGHEST) + bt_ref[...]
    pa_ref[...] = jnp.dot(at_ref[...], wa_ref[...],
                          preferred_element_type=jnp.float32,
                          precision=lax.Precision.HIGHEST) + ba_ref[...]


def _project_tables(time_table, act_table, W_time, b_time, W_act, b_act):
    V, D = time_table.shape
    grid = V // _ROW_BLOCK
    row_spec = pl.BlockSpec((_ROW_BLOCK, D), lambda i: (i, 0))
    full_w = pl.BlockSpec((D, D), lambda i: (0, 0))
    full_b = pl.BlockSpec((1, D), lambda i: (0, 0))
    out_shape = jax.ShapeDtypeStruct((V, D), jnp.float32)
    return pl.pallas_call(
        _project_body,
        grid=(grid,),
        in_specs=[row_spec, row_spec, full_w, full_b, full_w, full_b],
        out_specs=[row_spec, row_spec],
        out_shape=[out_shape, out_shape],
    )(time_table, act_table, W_time, b_time.reshape(1, D),
      W_act, b_act.reshape(1, D))


def _ctx_body(ctx_ref, g_ref, a_ref, o_ref):
    c = ctx_ref[...]
    gv = c[:, 0:1]
    av = c[:, 1:2]
    n = c.shape[0]
    oh_g = (lax.broadcasted_iota(jnp.int32, (n, g_ref.shape[0]), 1)
            == gv).astype(jnp.float32)
    oh_a = (lax.broadcasted_iota(jnp.int32, (n, a_ref.shape[0]), 1)
            == av).astype(jnp.float32)
    g_emb = jnp.dot(oh_g, g_ref[...], preferred_element_type=jnp.float32,
                    precision=lax.Precision.HIGHEST)
    a_emb = jnp.dot(oh_a, a_ref[...], preferred_element_type=jnp.float32,
                    precision=lax.Precision.HIGHEST)
    o_ref[...] = jnp.concatenate([g_emb, a_emb], axis=-1)


def _ctx_embed(context_tokens, gender_table, age_table):
    n = context_tokens.shape[0]
    dg = gender_table.shape[1]
    da = age_table.shape[1]
    return pl.pallas_call(
        _ctx_body,
        out_shape=jax.ShapeDtypeStruct((n, dg + da), jnp.float32),
    )(context_tokens, gender_table, age_table)


def _sc_gather(p_time, p_act, t_idx, a_idx):
    V, D = p_time.shape
    n_idx = t_idx.shape[1]
    mesh = plsc.VectorSubcoreMesh(core_axis_name="c", subcore_axis_name="s")
    out_t = jax.ShapeDtypeStruct((n_idx, D), jnp.float32)

    @functools.partial(
        pl.kernel, mesh=mesh, out_type=[out_t, out_t],
        compiler_params=pltpu.CompilerParams(use_tc_tiling_on_sc=False))
    def k(pt_hbm, pa_hbm, ti_hbm, ai_hbm, ot_hbm, oa_hbm):
        def body(ti_v, ai_v, ot_v, oa_v):
            pltpu.sync_copy(pt_hbm.at[ti_v.at[0]], ot_v)
            pltpu.sync_copy(pa_hbm.at[ai_v.at[0]], oa_v)

        pltpu.emit_pipeline(
            body,
            grid=(n_idx // _GATHER_W,),
            in_specs=[pl.BlockSpec((1, _GATHER_W), lambda i: (0, i)),
                      pl.BlockSpec((1, _GATHER_W), lambda i: (0, i))],
            out_specs=[pl.BlockSpec((_GATHER_W, D), lambda i: (i, 0)),
                       pl.BlockSpec((_GATHER_W, D), lambda i: (i, 0))],
            core_axis_name=("c", "s"),
            dimension_semantics=(pltpu.PARALLEL,),
        )(ti_hbm, ai_hbm, ot_hbm, oa_hbm)

    return k(p_time, p_act, t_idx, a_idx)


def kernel(context_tokens, time_tokens, act_tokens, time_table, act_table,
           age_table, gender_table, W_time, b_time, W_act, b_act):
    B, L = time_tokens.shape
    D = time_table.shape[1]
    t_idx = time_tokens.astype(jnp.int32).reshape(1, B * L)
    a_idx = act_tokens.astype(jnp.int32).reshape(1, B * L)

    p_time, p_act = _project_tables(time_table, act_table,
                                    W_time, b_time, W_act, b_act)
    ctx_emb = _ctx_embed(context_tokens.astype(jnp.int32),
                         gender_table, age_table)
    t_flat, a_flat = _sc_gather(p_time, p_act, t_idx, a_idx)
    return ctx_emb, t_flat.reshape(B, L, D), a_flat.reshape(B, L, D)
